# baseline (speedup 1.0000x reference)
import jax
import jax.numpy as jnp
from jax import lax
from jax.experimental import pallas as pl
from jax.experimental.pallas import tpu as pltpu
from jax.experimental.pallas import tpu_sc as plsc

D_MODEL = 1024
SEQ = 2048
BATCH = 4
NUM_CORES = 2
NUM_SUBCORES = 16
NUM_WORKERS = NUM_CORES * NUM_SUBCORES
SEQ_PER_WORKER = SEQ // NUM_WORKERS
CHUNK = 32
NUM_CHUNKS = SEQ_PER_WORKER // CHUNK
NUM_ITEMS = NUM_CHUNKS * BATCH
LANES = 16
VECS_PER_ROW = D_MODEL // LANES


def _body(x_hbm, emb_hbm, pe_hbm, out_hbm,
          idx0, idx1, ebuf, xbuf0, xbuf1,
          gsem, isem0, isem1, osem0, osem1):
    idx = [idx0, idx1]
    xbuf = [xbuf0, xbuf1]
    isem = [isem0, isem1]
    osem = [osem0, osem1]

    wid = lax.axis_index("s") * NUM_CORES + lax.axis_index("c")
    base = wid * SEQ_PER_WORKER

    def start_gather(c):
        pltpu.sync_copy(pe_hbm.at[pl.ds(base + c * CHUNK, CHUNK)], idx[c % 2])
        return pltpu.async_copy(emb_hbm.at[idx[c % 2]], ebuf, gsem)

    def start_in(k):
        c, b = k // BATCH, k % BATCH
        return pltpu.async_copy(
            x_hbm.at[b, pl.ds(base + c * CHUNK, CHUNK)], xbuf[k % 2],
            isem[k % 2])

    g_desc = start_gather(0)
    in_desc = [None, None]
    out_desc = [None, None]
    in_desc[0] = start_in(0)

    for k in range(NUM_ITEMS):
        c, b = k // BATCH, k % BATCH
        cur = k % 2
        if k + 1 < NUM_ITEMS:
            nxt = (k + 1) % 2
            if out_desc[nxt] is not None:
                out_desc[nxt].wait()
                out_desc[nxt] = None
            in_desc[nxt] = start_in(k + 1)
        in_desc[cur].wait()
        if b == 0:
            g_desc.wait()
        xb = xbuf[cur]

        def strip_add(i, _):
            r = i // 32
            h = (i % 32) * (VECS_PER_ROW // 32)
            for o in range(VECS_PER_ROW // 32):
                plsc.addupdate(
                    xb.at[r, pl.ds((h + o) * LANES, LANES)],
                    ebuf[r, pl.ds((h + o) * LANES, LANES)],
                )
            return 0

        lax.fori_loop(0, CHUNK * 32, strip_add, 0)
        if b == BATCH - 1 and c + 1 < NUM_CHUNKS:
            g_desc = start_gather(c + 1)
        out_desc[cur] = pltpu.async_copy(
            xb, out_hbm.at[b, pl.ds(base + c * CHUNK, CHUNK)], osem[cur])

    for d in out_desc:
        if d is not None:
            d.wait()


def kernel(x, emb, pe):
    mesh = plsc.VectorSubcoreMesh(
        core_axis_name="c",
        subcore_axis_name="s",
        num_cores=NUM_CORES,
        num_subcores=NUM_SUBCORES,
    )
    run = pl.kernel(
        _body,
        out_type=jax.ShapeDtypeStruct((BATCH, SEQ, D_MODEL), jnp.float32),
        mesh=mesh,
        scratch_types=[
            pltpu.VMEM((CHUNK,), jnp.int32),
            pltpu.VMEM((CHUNK,), jnp.int32),
            pltpu.VMEM((CHUNK, D_MODEL), jnp.float32),
            pltpu.VMEM((CHUNK, D_MODEL), jnp.float32),
            pltpu.VMEM((CHUNK, D_MODEL), jnp.float32),
            pltpu.SemaphoreType.DMA,
            pltpu.SemaphoreType.DMA,
            pltpu.SemaphoreType.DMA,
            pltpu.SemaphoreType.DMA,
            pltpu.SemaphoreType.DMA,
        ],
        name="learnt_pos_enc_sc",
    )
    return run(x, emb, pe)


# speedup vs baseline: 1.2575x; 1.2575x over previous
import jax
import jax.numpy as jnp
from jax import lax
from jax.experimental import pallas as pl
from jax.experimental.pallas import tpu as pltpu
from jax.experimental.pallas import tpu_sc as plsc

D_MODEL = 1024
SEQ = 2048
BATCH = 4
NUM_CORES = 2
NUM_SUBCORES = 16
NUM_WORKERS = NUM_CORES * NUM_SUBCORES
SEQ_PER_WORKER = SEQ // NUM_WORKERS
CHUNK = 32
NUM_CHUNKS = SEQ_PER_WORKER // CHUNK
NUM_ITEMS = NUM_CHUNKS * BATCH
LANES = 16
VECS_PER_ROW = D_MODEL // LANES


def _body(x_hbm, emb_hbm, pe_hbm, out_hbm,
          idx0, idx1, ebuf, xbuf0, xbuf1,
          gsem, isem0, isem1, osem0, osem1):
    idx = [idx0, idx1]
    xbuf = [xbuf0, xbuf1]
    isem = [isem0, isem1]
    osem = [osem0, osem1]

    wid = lax.axis_index("s") * NUM_CORES + lax.axis_index("c")
    base = wid * SEQ_PER_WORKER

    def start_gather(c):
        pltpu.sync_copy(pe_hbm.at[pl.ds(base + c * CHUNK, CHUNK)], idx[c % 2])
        return pltpu.async_copy(emb_hbm.at[idx[c % 2]], ebuf, gsem)

    def start_in(k):
        c, b = k // BATCH, k % BATCH
        return pltpu.async_copy(
            x_hbm.at[b, pl.ds(base + c * CHUNK, CHUNK)], xbuf[k % 2],
            isem[k % 2])

    g_desc = start_gather(0)
    in_desc = [None, None]
    out_desc = [None, None]
    in_desc[0] = start_in(0)

    for k in range(NUM_ITEMS):
        c, b = k // BATCH, k % BATCH
        cur = k % 2
        if k + 1 < NUM_ITEMS:
            nxt = (k + 1) % 2
            if out_desc[nxt] is not None:
                out_desc[nxt].wait()
                out_desc[nxt] = None
            in_desc[nxt] = start_in(k + 1)
        in_desc[cur].wait()
        if b == 0:
            g_desc.wait()
        xb = xbuf[cur]

        def strip_add(i, _):
            r = i // 16
            h = (i % 16) * (VECS_PER_ROW // 16)
            for o in range(VECS_PER_ROW // 16):
                plsc.addupdate(
                    xb.at[r, pl.ds((h + o) * LANES, LANES)],
                    ebuf[r, pl.ds((h + o) * LANES, LANES)],
                )
            return 0

        lax.fori_loop(0, CHUNK * 16, strip_add, 0)
        if b == BATCH - 1 and c + 1 < NUM_CHUNKS:
            g_desc = start_gather(c + 1)
        out_desc[cur] = pltpu.async_copy(
            xb, out_hbm.at[b, pl.ds(base + c * CHUNK, CHUNK)], osem[cur])

    for d in out_desc:
        if d is not None:
            d.wait()


def kernel(x, emb, pe):
    mesh = plsc.VectorSubcoreMesh(
        core_axis_name="c",
        subcore_axis_name="s",
        num_cores=NUM_CORES,
        num_subcores=NUM_SUBCORES,
    )
    run = pl.kernel(
        _body,
        out_type=jax.ShapeDtypeStruct((BATCH, SEQ, D_MODEL), jnp.float32),
        mesh=mesh,
        scratch_types=[
            pltpu.VMEM((CHUNK,), jnp.int32),
            pltpu.VMEM((CHUNK,), jnp.int32),
            pltpu.VMEM((CHUNK, D_MODEL), jnp.float32),
            pltpu.VMEM((CHUNK, D_MODEL), jnp.float32),
            pltpu.VMEM((CHUNK, D_MODEL), jnp.float32),
            pltpu.SemaphoreType.DMA,
            pltpu.SemaphoreType.DMA,
            pltpu.SemaphoreType.DMA,
            pltpu.SemaphoreType.DMA,
            pltpu.SemaphoreType.DMA,
        ],
        name="learnt_pos_enc_sc",
    )
    return run(x, emb, pe)
